# baseline (speedup 1.0000x reference)
import functools

import jax
import jax.numpy as jnp
from jax import lax
from jax.experimental import pallas as pl
from jax.experimental.pallas import tpu as pltpu
from jax.experimental.pallas import tpu_sc as plsc

_NC = 2
_NS = 16
_NW = _NC * _NS

_D = 64
_P = 2 * _D
_CH = 112
_NB = 4


def _row(tiles_v, off_v, pos, t, iotas):
    osp = plsc.load_gather(off_v, [jnp.full((16,), pos, jnp.int32)])
    tsp = jnp.full((16,), t, jnp.int32)
    return [plsc.load_gather(tiles_v, [tsp, osp + iotas[j]])
            for j in range(4)]


def _sc_body(nchunk, hpw, tt, toff, ht, hoff, table, gath, part,
             tidx_v, toff_v, hidx_v, hoff_v, hbuf_v, hout_v, pout_v,
             *rest):
    tiles = rest[:_NB]
    hsem = rest[_NB]
    sems = rest[_NB + 1:]
    cid = lax.axis_index("c")
    sid = lax.axis_index("s")
    wid = sid * _NC + cid
    ntok = nchunk * _CH

    iotas = [lax.iota(jnp.int32, 16) + 16 * j for j in range(4)]

    pltpu.sync_copy(tt.at[pl.ds(wid * ntok, ntok)], tidx_v)
    pltpu.sync_copy(toff.at[pl.ds(wid * ntok, ntok)], toff_v)
    pltpu.sync_copy(ht.at[pl.ds(wid * hpw, hpw)], hidx_v)
    pltpu.sync_copy(hoff.at[pl.ds(wid * hpw, hpw)], hoff_v)

    head_cp = pltpu.async_copy(table.at[hidx_v], hbuf_v, hsem)

    def start(ci, bi):
        pltpu.async_copy(table.at[tidx_v.at[pl.ds(ci * _CH, _CH)]],
                         tiles[bi], sems[bi])

    def wait(ci, bi):
        pltpu.make_async_copy(table.at[tidx_v.at[pl.ds(ci * _CH, _CH)]],
                              tiles[bi], sems[bi]).wait()

    def acc_chunk(ci, bi, accs):
        base = ci * _CH

        def tok_step(t, a):
            row = _row(tiles[bi], toff_v, base + t, t, iotas)
            return tuple(a[j] + row[j] for j in range(4))

        return lax.fori_loop(0, _CH, tok_step, accs, unroll=8)

    for b in range(_NB):
        start(b, b)

    def pipe_step(p, accs):
        c = _NB * p
        for b in range(_NB):
            wait(c + b, b)
            accs = acc_chunk(c + b, b, accs)
            start(c + _NB + b, b)
        return accs

    zero = jnp.zeros((16,), jnp.float32)
    accs = lax.fori_loop(0, nchunk // _NB - 1, pipe_step,
                         (zero, zero, zero, zero))
    for b in range(_NB):
        wait(nchunk - _NB + b, b)
        accs = acc_chunk(nchunk - _NB + b, b, accs)

    for j in range(4):
        pout_v[pl.ds(16 * j, 16)] = accs[j]
    pltpu.sync_copy(pout_v, part.at[pl.ds(wid * _D, _D)])

    head_cp.wait()

    def head_step(t, carry):
        row = _row(hbuf_v, hoff_v, t, t, iotas)
        for j in range(4):
            hout_v[pl.ds(t * _D + 16 * j, 16)] = row[j]
        return carry

    lax.fori_loop(0, hpw, head_step, 0, unroll=8)
    pltpu.sync_copy(hout_v, gath.at[pl.ds(wid * hpw * _D, hpw * _D)])


def _sc_gather(tt, toff, ht, hoff, table2, b, nchunk, hpw):
    ntok = nchunk * _CH
    mesh = plsc.VectorSubcoreMesh(core_axis_name="c", subcore_axis_name="s")
    fn = functools.partial(
        pl.kernel,
        out_type=[
            jax.ShapeDtypeStruct((b * _D,), jnp.float32),
            jax.ShapeDtypeStruct((_NW * _D,), jnp.float32),
        ],
        mesh=mesh,
        compiler_params=pltpu.CompilerParams(use_tc_tiling_on_sc=True,
                                             needs_layout_passes=False),
        scratch_types=[
            pltpu.VMEM((ntok,), jnp.int32),
            pltpu.VMEM((ntok,), jnp.int32),
            pltpu.VMEM((hpw,), jnp.int32),
            pltpu.VMEM((hpw,), jnp.int32),
            pltpu.VMEM((hpw, _P), jnp.float32),
            pltpu.VMEM((hpw * _D,), jnp.float32),
            pltpu.VMEM((_D,), jnp.float32),
        ] + [pltpu.VMEM((_CH, _P), jnp.float32)] * _NB
          + [pltpu.SemaphoreType.DMA] * (1 + _NB),
    )(functools.partial(_sc_body, nchunk, hpw))
    return fn(tt, toff, ht, hoff, table2)


_RB = 2000


def _repack_body(lo_ref, hi_ref, out_ref):
    out_ref[:, :_D] = lo_ref[...]
    out_ref[:, _D:] = hi_ref[...]


def _tc_repack(emb_weight):
    v = emb_weight.shape[0]
    nblk = v // 2 // _RB

    def lo_map(i):
        return (i, 0)

    def hi_map(i):
        return (i + nblk, 0)

    return pl.pallas_call(
        _repack_body,
        grid=(nblk,),
        in_specs=[pl.BlockSpec((_RB, _D), lo_map),
                  pl.BlockSpec((_RB, _D), hi_map)],
        out_specs=pl.BlockSpec((_RB, _P), lo_map),
        out_shape=jax.ShapeDtypeStruct((v // 2, _P), jnp.float32),
    )(emb_weight, emb_weight)


def _tc_body(tail_count, b, g_ref, p_ref, w_ref, b_ref, o_ref):
    psum = jnp.sum(p_ref[...], axis=0)
    last = (g_ref[b - 1, :] + psum) * (1.0 / tail_count)
    rows = lax.broadcasted_iota(jnp.int32, (b, 1), 0)
    mean = jnp.where(rows == b - 1, last[None, :], g_ref[...])
    o_ref[...] = lax.dot_general(
        mean, w_ref[...], (((1,), (1,)), ((), ())),
        preferred_element_type=jnp.float32) + b_ref[...]


def _tc_classify(gathered, partials, fc_w, fc_b2d, tail_count):
    b = gathered.shape[0]
    nc = fc_w.shape[0]
    return pl.pallas_call(
        functools.partial(_tc_body, tail_count, b),
        out_shape=jax.ShapeDtypeStruct((b, nc), jnp.float32),
    )(gathered, partials, fc_w, fc_b2d)


def kernel(text, off, emb_weight, fc_w, fc_b):
    t = text.shape[0]
    b = off.shape[0]
    tail = t - b
    tail_count = t - b + 1
    assert b % _NW == 0 and tail % (_NW * _CH * _NB) == 0
    hpw = b // _NW
    nchunk = tail // (_NW * _CH)

    vhalf = emb_weight.shape[0] // 2
    in_hi = text >= vhalf
    pair = jnp.where(in_hi, text - vhalf, text)
    half = jnp.where(in_hi, _D, 0).astype(jnp.int32)
    tt = pair[b:]
    toff = half[b:]
    ht = pair[:b]
    hoff = half[:b]
    table2 = _tc_repack(emb_weight)
    gathered, partials = _sc_gather(tt, toff, ht, hoff, table2, b, nchunk,
                                    hpw)
    return _tc_classify(gathered.reshape(b, _D), partials.reshape(_NW, _D),
                        fc_w, fc_b.reshape(1, -1), float(tail_count))


# speedup vs baseline: 1.2140x; 1.2140x over previous
import functools

import jax
import jax.numpy as jnp
from jax import lax
from jax.experimental import pallas as pl
from jax.experimental.pallas import tpu as pltpu
from jax.experimental.pallas import tpu_sc as plsc

_NC = 2
_NS = 16
_NW = _NC * _NS

_D = 64
_P = 2 * _D
_CH = 112
_NB = 4


def _row(tiles_v, off_v, pos, t, iotas):
    osp = plsc.load_gather(off_v, [jnp.full((16,), pos, jnp.int32)])
    tsp = jnp.full((16,), t, jnp.int32)
    return [plsc.load_gather(tiles_v, [tsp, osp + iotas[j]])
            for j in range(4)]


def _sc_body(nchunk, hpw, tt, toff, ht, hoff, table, gath, part,
             tidx_v, toff_v, hidx_v, hoff_v, hbuf_v, hout_v, pout_v,
             *rest):
    tiles = rest[:_NB]
    hsem = rest[_NB]
    sems = rest[_NB + 1:]
    cid = lax.axis_index("c")
    sid = lax.axis_index("s")
    wid = sid * _NC + cid
    ntok = nchunk * _CH

    iotas = [lax.iota(jnp.int32, 16) + 16 * j for j in range(4)]

    pltpu.sync_copy(tt.at[pl.ds(wid * ntok, ntok)], tidx_v)
    pltpu.sync_copy(toff.at[pl.ds(wid * ntok, ntok)], toff_v)
    pltpu.sync_copy(ht.at[pl.ds(wid * hpw, hpw)], hidx_v)
    pltpu.sync_copy(hoff.at[pl.ds(wid * hpw, hpw)], hoff_v)

    head_cp = pltpu.async_copy(table.at[hidx_v], hbuf_v, hsem)

    def start(ci, bi):
        pltpu.async_copy(table.at[tidx_v.at[pl.ds(ci * _CH, _CH)]],
                         tiles[bi], sems[bi])

    def wait(ci, bi):
        pltpu.make_async_copy(table.at[tidx_v.at[pl.ds(ci * _CH, _CH)]],
                              tiles[bi], sems[bi]).wait()

    def acc_chunk(ci, bi, accs):
        base = ci * _CH

        def tok_step(t, a):
            row = _row(tiles[bi], toff_v, base + t, t, iotas)
            return tuple(a[j] + row[j] for j in range(4))

        return lax.fori_loop(0, _CH, tok_step, accs, unroll=8)

    for b in range(_NB):
        start(b, b)

    def pipe_step(p, accs):
        c = _NB * p
        for b in range(_NB):
            wait(c + b, b)
            accs = acc_chunk(c + b, b, accs)
            start(c + _NB + b, b)
        return accs

    zero = jnp.zeros((16,), jnp.float32)
    accs = lax.fori_loop(0, nchunk // _NB - 1, pipe_step,
                         (zero, zero, zero, zero))
    for b in range(_NB):
        wait(nchunk - _NB + b, b)
        accs = acc_chunk(nchunk - _NB + b, b, accs)

    for j in range(4):
        pout_v[pl.ds(16 * j, 16)] = accs[j]
    pltpu.sync_copy(pout_v, part.at[pl.ds(wid * _D, _D)])

    head_cp.wait()

    def head_step(t, carry):
        row = _row(hbuf_v, hoff_v, t, t, iotas)
        for j in range(4):
            hout_v[pl.ds(t * _D + 16 * j, 16)] = row[j]
        return carry

    lax.fori_loop(0, hpw, head_step, 0, unroll=8)
    pltpu.sync_copy(hout_v, gath.at[pl.ds(wid * hpw * _D, hpw * _D)])


def _sc_gather(tt, toff, ht, hoff, table2, b, nchunk, hpw):
    ntok = nchunk * _CH
    mesh = plsc.VectorSubcoreMesh(core_axis_name="c", subcore_axis_name="s")
    fn = functools.partial(
        pl.kernel,
        out_type=[
            jax.ShapeDtypeStruct((b * _D,), jnp.float32),
            jax.ShapeDtypeStruct((_NW * _D,), jnp.float32),
        ],
        mesh=mesh,
        compiler_params=pltpu.CompilerParams(use_tc_tiling_on_sc=True,
                                             needs_layout_passes=False),
        scratch_types=[
            pltpu.VMEM((ntok,), jnp.int32),
            pltpu.VMEM((ntok,), jnp.int32),
            pltpu.VMEM((hpw,), jnp.int32),
            pltpu.VMEM((hpw,), jnp.int32),
            pltpu.VMEM((hpw, _P), jnp.float32),
            pltpu.VMEM((hpw * _D,), jnp.float32),
            pltpu.VMEM((_D,), jnp.float32),
        ] + [pltpu.VMEM((_CH, _P), jnp.float32)] * _NB
          + [pltpu.SemaphoreType.DMA] * (1 + _NB),
    )(functools.partial(_sc_body, nchunk, hpw))
    return fn(tt, toff, ht, hoff, table2)


_RB = 2000


def _repack_body(in_ref, out_ref):
    out_ref[:, :_D] = in_ref[0]
    out_ref[:, _D:] = in_ref[1]


def _tc_repack(emb_weight):
    v = emb_weight.shape[0]
    nblk = v // 2 // _RB
    emb3 = emb_weight.reshape(2, v // 2, _D)
    return pl.pallas_call(
        _repack_body,
        grid=(nblk,),
        in_specs=[pl.BlockSpec((2, _RB, _D), lambda i: (0, i, 0))],
        out_specs=pl.BlockSpec((_RB, _P), lambda i: (i, 0)),
        out_shape=jax.ShapeDtypeStruct((v // 2, _P), jnp.float32),
    )(emb3)


def _tc_body(tail_count, b, g_ref, p_ref, w_ref, b_ref, o_ref):
    psum = jnp.sum(p_ref[...], axis=0)
    last = (g_ref[b - 1, :] + psum) * (1.0 / tail_count)
    rows = lax.broadcasted_iota(jnp.int32, (b, 1), 0)
    mean = jnp.where(rows == b - 1, last[None, :], g_ref[...])
    o_ref[...] = lax.dot_general(
        mean, w_ref[...], (((1,), (1,)), ((), ())),
        preferred_element_type=jnp.float32) + b_ref[...]


def _tc_classify(gathered, partials, fc_w, fc_b2d, tail_count):
    b = gathered.shape[0]
    nc = fc_w.shape[0]
    return pl.pallas_call(
        functools.partial(_tc_body, tail_count, b),
        out_shape=jax.ShapeDtypeStruct((b, nc), jnp.float32),
    )(gathered, partials, fc_w, fc_b2d)


def kernel(text, off, emb_weight, fc_w, fc_b):
    t = text.shape[0]
    b = off.shape[0]
    tail = t - b
    tail_count = t - b + 1
    assert b % _NW == 0 and tail % (_NW * _CH * _NB) == 0
    hpw = b // _NW
    nchunk = tail // (_NW * _CH)

    vhalf = emb_weight.shape[0] // 2
    in_hi = text >= vhalf
    pair = jnp.where(in_hi, text - vhalf, text)
    half = jnp.where(in_hi, _D, 0).astype(jnp.int32)
    tt = pair[b:]
    toff = half[b:]
    ht = pair[:b]
    hoff = half[:b]
    table2 = _tc_repack(emb_weight)
    gathered, partials = _sc_gather(tt, toff, ht, hoff, table2, b, nchunk,
                                    hpw)
    return _tc_classify(gathered.reshape(b, _D), partials.reshape(_NW, _D),
                        fc_w, fc_b.reshape(1, -1), float(tail_count))
